# Initial kernel scaffold; baseline (speedup 1.0000x reference)
import jax
import jax.numpy as jnp
from jax.experimental import pallas as pl


def kernel(input, table):
    raise NotImplementedError("write your pallas kernel here")


# speedup vs baseline: 6.0321x; 6.0321x over previous
import functools

import jax
import jax.numpy as jnp
from jax import lax
from jax.experimental import pallas as pl
from jax.experimental.pallas import tpu as pltpu
from jax.experimental.pallas import tpu_sc as plsc

VOCAB = 1000000
EMBED = 32
B, F, L = 4096, 26, 20

NC, NS = 2, 16
NW = NC * NS
SEGS = B * F
SEGS_PER_W = SEGS // NW
CHUNK_SEGS = 64
ROWS_PER_CHUNK = CHUNK_SEGS * L
GATHER_W = 128
N_GATHERS = ROWS_PER_CHUNK // GATHER_W
N_CHUNKS = SEGS_PER_W // CHUNK_SEGS


def _body(idx_hbm, table_hbm, out_hbm, idx_v, rows_v, outbuf, sem0, sem1):
    sems = (sem0, sem1)
    cid = lax.axis_index("c")
    sid = lax.axis_index("s")
    w = sid * NC + cid
    idx_base = w * (SEGS_PER_W * L)
    seg_base = w * SEGS_PER_W

    def load_and_fire(c, b):
        pltpu.sync_copy(
            idx_hbm.at[pl.ds(idx_base + c * ROWS_PER_CHUNK, ROWS_PER_CHUNK)],
            idx_v.at[b],
        )
        for j in range(N_GATHERS):
            pltpu.async_copy(
                table_hbm.at[idx_v.at[b, pl.ds(j * GATHER_W, GATHER_W)]],
                rows_v.at[b, pl.ds(j * GATHER_W, GATHER_W)],
                sems[b],
            )

    def drain(b):
        pltpu.make_async_copy(
            table_hbm.at[pl.ds(0, ROWS_PER_CHUNK)], rows_v.at[b], sems[b]
        ).wait()

    def compute(c, b):
        def seg_body(s, carry):
            r0 = s * L
            acc0 = rows_v[b, r0, pl.ds(0, 16)]
            acc1 = rows_v[b, r0, pl.ds(16, 16)]
            for j in range(1, L):
                acc0 = acc0 + rows_v[b, r0 + j, pl.ds(0, 16)]
                acc1 = acc1 + rows_v[b, r0 + j, pl.ds(16, 16)]
            outbuf[b, s, pl.ds(0, 16)] = acc0
            outbuf[b, s, pl.ds(16, 16)] = acc1
            return carry

        lax.fori_loop(0, CHUNK_SEGS, seg_body, 0)
        pltpu.sync_copy(
            outbuf.at[b],
            out_hbm.at[pl.ds(seg_base + c * CHUNK_SEGS, CHUNK_SEGS)],
        )

    load_and_fire(0, 0)
    load_and_fire(1, 1)

    def outer(g, carry):
        for b in range(2):
            c = 2 * g + b
            drain(b)
            compute(c, b)

            @pl.when(c + 2 < N_CHUNKS)
            def _():
                load_and_fire(c + 2, b)

        return carry

    lax.fori_loop(0, N_CHUNKS // 2, outer, 0)


@jax.jit
def _emb(idx2, table):
    mesh = plsc.VectorSubcoreMesh(
        core_axis_name="c", subcore_axis_name="s", num_cores=NC, num_subcores=NS
    )
    f = pl.kernel(
        _body,
        out_type=jax.ShapeDtypeStruct((SEGS, EMBED), jnp.float32),
        mesh=mesh,
        scratch_types=[
            pltpu.VMEM((2, ROWS_PER_CHUNK), jnp.int32),
            pltpu.VMEM((2, ROWS_PER_CHUNK, EMBED), jnp.float32),
            pltpu.VMEM((2, CHUNK_SEGS, EMBED), jnp.float32),
            pltpu.SemaphoreType.DMA,
            pltpu.SemaphoreType.DMA,
        ],
        compiler_params=pltpu.CompilerParams(use_tc_tiling_on_sc=False),
    )
    return f(idx2, table)


def kernel(input, table):
    idx = input.reshape(SEGS * L)
    out = _emb(idx, table)
    return out.reshape(B, F, EMBED)


# speedup vs baseline: 6.0395x; 1.0012x over previous
import functools

import jax
import jax.numpy as jnp
from jax import lax
from jax.experimental import pallas as pl
from jax.experimental.pallas import tpu as pltpu
from jax.experimental.pallas import tpu_sc as plsc

VOCAB = 1000000
EMBED = 32
B, F, L = 4096, 26, 20

NC, NS = 2, 16
NW = NC * NS
SEGS = B * F
SEGS_PER_W = SEGS // NW
CHUNK_SEGS = 64
ROWS_PER_CHUNK = CHUNK_SEGS * L
GATHER_W = 128
N_GATHERS = ROWS_PER_CHUNK // GATHER_W
N_CHUNKS = SEGS_PER_W // CHUNK_SEGS


def _body(idx_hbm, table_hbm, out_hbm, idx_v, rows_v, outbuf, sem0, sem1):
    sems = (sem0, sem1)
    cid = lax.axis_index("c")
    sid = lax.axis_index("s")
    w = sid * NC + cid
    idx_base = w * (SEGS_PER_W * L)
    seg_base = w * SEGS_PER_W

    def load_and_fire(c, b):
        pltpu.sync_copy(
            idx_hbm.at[pl.ds(idx_base + c * ROWS_PER_CHUNK, ROWS_PER_CHUNK)],
            idx_v.at[b],
        )
        for j in range(N_GATHERS):
            pltpu.async_copy(
                table_hbm.at[idx_v.at[b, pl.ds(j * GATHER_W, GATHER_W)]],
                rows_v.at[b, pl.ds(j * GATHER_W, GATHER_W)],
                sems[b],
            )

    def drain(b):
        pltpu.make_async_copy(
            table_hbm.at[pl.ds(0, ROWS_PER_CHUNK)], rows_v.at[b], sems[b]
        ).wait()

    def compute(c, b):
        def seg_body(s, carry):
            r0 = s * L
            acc0 = rows_v[b, r0, pl.ds(0, 16)]
            acc1 = rows_v[b, r0, pl.ds(16, 16)]
            for j in range(1, L):
                acc0 = acc0 + rows_v[b, r0 + j, pl.ds(0, 16)]
                acc1 = acc1 + rows_v[b, r0 + j, pl.ds(16, 16)]
            outbuf[b, s, pl.ds(0, 16)] = acc0
            outbuf[b, s, pl.ds(16, 16)] = acc1
            return carry

        lax.fori_loop(0, CHUNK_SEGS, seg_body, 0)
        pltpu.sync_copy(
            outbuf.at[b],
            out_hbm.at[pl.ds(seg_base + c * CHUNK_SEGS, CHUNK_SEGS)],
        )

    load_and_fire(0, 0)
    load_and_fire(1, 1)

    def outer(g, carry):
        for b in range(2):
            c = 2 * g + b
            drain(b)
            compute(c, b)

            @pl.when(c + 2 < N_CHUNKS)
            def _():
                load_and_fire(c + 2, b)

        return carry

    lax.fori_loop(0, N_CHUNKS // 2, outer, 0)


@jax.jit
def _emb(idx2, t128):
    table = t128.reshape(VOCAB, EMBED)
    mesh = plsc.VectorSubcoreMesh(
        core_axis_name="c", subcore_axis_name="s", num_cores=NC, num_subcores=NS
    )
    f = pl.kernel(
        _body,
        out_type=jax.ShapeDtypeStruct((SEGS, EMBED), jnp.float32),
        mesh=mesh,
        scratch_types=[
            pltpu.VMEM((2, ROWS_PER_CHUNK), jnp.int32),
            pltpu.VMEM((2, ROWS_PER_CHUNK, EMBED), jnp.float32),
            pltpu.VMEM((2, CHUNK_SEGS, EMBED), jnp.float32),
            pltpu.SemaphoreType.DMA,
            pltpu.SemaphoreType.DMA,
        ],
        compiler_params=pltpu.CompilerParams(use_tc_tiling_on_sc=False),
    )
    return f(idx2, table)


def kernel(input, table):
    idx = input.reshape(SEGS * L)
    t128 = table.reshape(VOCAB * EMBED // 128, 128)
    out = _emb(idx, t128)
    return out.reshape(B, F, EMBED)
